# baseline (speedup 1.0000x reference)
import jax
import jax.numpy as jnp
from jax import lax
from jax.experimental import pallas as pl
from jax.experimental.pallas import tpu as pltpu


def _dot(a, b, ca, cb):
    return lax.dot_general(a, b, ((( ca,), (cb,)), ((), ())),
                           precision=lax.Precision.HIGHEST,
                           preferred_element_type=jnp.float32)


def _bf(x):
    return x.astype(jnp.bfloat16).astype(jnp.float32)


def _branch(LT, KT, xT, We1T, be1, We2T, be2, Wn1T, bn1, Wn2T, bn2, n):
    xTb = _bf(xT)
    cs2 = _dot(xTb, _bf(We1T[:, 0:3]), 0, 1)
    baseT = _dot(_bf(We1T[:, 3:6]), xTb, 1, 0) + be1
    wlp = jnp.broadcast_to(_bf(We1T[:, 6:7]), baseT.shape)
    wkp = jnp.broadcast_to(_bf(We1T[:, 7:8]), baseT.shape)
    a3 = (LT[:, None, :] * wlp[None] + KT[:, None, :] * wkp[None]
          + cs2[:, :, None] + baseT[None])
    agg0T = _bf(jnp.maximum(a3, 0.0)).sum(axis=0)
    aggT = _dot(_bf(We2T), agg0T, 1, 0) + n * be2
    gT = jnp.maximum(_dot(_bf(Wn1T[:, 0:3]), xTb, 1, 0)
                     + _dot(_bf(Wn1T[:, 3:]), _bf(aggT), 1, 0) + bn1, 0.0)
    return _dot(_bf(Wn2T), _bf(gT), 1, 0) + bn2


def _body(LT_ref, KT_ref, qT_ref, dqT_ref, pT_ref, dpT_ref, mT_ref,
          We1Tq_ref, be1q_ref, We2Tq_ref, be2q_ref,
          Wn1Tq_ref, bn1q_ref, Wn2Tq_ref, bn2q_ref,
          We1Tp_ref, be1p_ref, We2Tp_ref, be2p_ref,
          Wn1Tp_ref, bn1p_ref, Wn2Tp_ref, bn2p_ref,
          hqT_ref, hpT_ref):
    LT = _bf(LT_ref[0])
    KT = _bf(KT_ref[0])
    mrow = mT_ref[0]
    n = LT.shape[0]
    s_count = qT_ref.shape[1]
    for s in range(s_count):
        xTq = jnp.concatenate([qT_ref[0, s:s + 1, :], dqT_ref[0, s:s + 1, :],
                               mrow], axis=0)
        xTp = jnp.concatenate([pT_ref[0, s:s + 1, :], dpT_ref[0, s:s + 1, :],
                               mrow], axis=0)
        outq = _branch(LT, KT, xTq, We1Tq_ref[...], be1q_ref[...],
                       We2Tq_ref[...], be2q_ref[...], Wn1Tq_ref[...],
                       bn1q_ref[...], Wn2Tq_ref[...], bn2q_ref[...], n)
        outp = _branch(LT, KT, xTp, We1Tp_ref[...], be1p_ref[...],
                       We2Tp_ref[...], be2p_ref[...], Wn1Tp_ref[...],
                       bn1p_ref[...], Wn2Tp_ref[...], bn2p_ref[...], n)
        hqT_ref[0, s, :] = outq[0]
        hpT_ref[0, s, :] = outp[0]


def _kernel_tc(q, p, dq, dp, m, t, dt, length, k,
               Wqe1, bqe1, Wqe2, bqe2, Wqn1, bqn1, Wqn2, bqn2,
               Wpe1, bpe1, Wpe2, bpe2, Wpn1, bpn1, Wpn2, bpn2):
    B, N, S = q.shape
    H = Wqe1.shape[1]
    f32 = jnp.float32

    LT = jnp.swapaxes(length, 1, 2)
    KT = jnp.swapaxes(k, 1, 2)
    qT = jnp.swapaxes(q, 1, 2)
    dqT = jnp.swapaxes(dq, 1, 2)
    pT = jnp.swapaxes(p, 1, 2)
    dpT = jnp.swapaxes(dp, 1, 2)
    mT = jnp.swapaxes(m, 1, 2)

    wargs = (Wqe1.T, bqe1[:, None], Wqe2.T, bqe2[:, None],
             Wqn1.T, bqn1[:, None], Wqn2.T, bqn2[:, None],
             Wpe1.T, bpe1[:, None], Wpe2.T, bpe2[:, None],
             Wpn1.T, bpn1[:, None], Wpn2.T, bpn2[:, None])

    def bspec(shape3):
        return pl.BlockSpec(shape3, lambda b: (b, 0, 0))

    def wspec(arr):
        sh = arr.shape
        return pl.BlockSpec(sh, lambda b: tuple(0 for _ in sh))

    grid_spec = pl.GridSpec(
        grid=(B,),
        in_specs=[bspec((1, N, N)), bspec((1, N, N)),
                  bspec((1, S, N)), bspec((1, S, N)),
                  bspec((1, S, N)), bspec((1, S, N)),
                  bspec((1, 1, N))] + [wspec(w) for w in wargs],
        out_specs=[bspec((1, S, N)), bspec((1, S, N))],
    )

    hqT, hpT = pl.pallas_call(
        _body,
        grid_spec=grid_spec,
        out_shape=[jax.ShapeDtypeStruct((B, S, N), f32),
                   jax.ShapeDtypeStruct((B, S, N), f32)],
        compiler_params=pltpu.CompilerParams(
            dimension_semantics=("arbitrary",)),
    )(LT, KT, qT, dqT, pT, dpT, mT, *wargs)

    return jnp.swapaxes(hqT, 1, 2), jnp.swapaxes(hpT, 1, 2)


import functools
from jax.experimental.pallas import tpu_sc as plsc

_LANES = 16


def _rnd1(x):
    u = jax.lax.bitcast_convert_type(x, jnp.int32)
    lsb = jax.lax.shift_right_logical(u, 16) & 1
    r = (u + 32767 + lsb) & jnp.int32(-65536)
    return jax.lax.bitcast_convert_type(r, jnp.float32)


def _rnd2(a, b):
    return _rnd1(a), _rnd1(b)


def _gat(vec, idxv):
    return vec.at[idxv].get(mode="promise_in_bounds")


def _round_region(ref, nchunks):
    def body(c, carry):
        o = c * 32
        a = ref[pl.ds(o, 16)]
        b = ref[pl.ds(o + 16, 16)]
        ar, br = _rnd2(a, b)
        ref[pl.ds(o, 16)] = ar
        ref[pl.ds(o + 16, 16)] = br
        return carry
    jax.lax.fori_loop(0, nchunks // 2, body, 0)


def _sc_call(B, N, S, f32, args):
    NSB = 4
    mesh = plsc.VectorSubcoreMesh(core_axis_name="c", subcore_axis_name="s")
    NW = 32
    BPW = B // NW

    scratch = [
        pltpu.VMEM((N * N,), f32),
        pltpu.VMEM((N * N,), f32),
        pltpu.VMEM((S * N,), f32),
        pltpu.VMEM((S * N,), f32),
        pltpu.VMEM((S * N,), f32),
        pltpu.VMEM((S * N,), f32),
        pltpu.VMEM((N,), f32),
        pltpu.VMEM((NSB * N * 32,), f32),
        pltpu.VMEM((NSB * N * 32,), f32),
        pltpu.VMEM((NSB * N * 32,), f32),
        pltpu.VMEM((512,), f32),
        pltpu.VMEM((64,), f32),
        pltpu.VMEM((2048,), f32),
        pltpu.VMEM((64,), f32),
        pltpu.VMEM((2240,), f32),
        pltpu.VMEM((64,), f32),
        pltpu.VMEM((64,), f32),
        pltpu.VMEM((32,), f32),
        pltpu.VMEM((S * N,), f32),
        pltpu.VMEM((S * N,), f32),
    ]

    @functools.partial(
        pl.kernel, mesh=mesh,
        out_type=[jax.ShapeDtypeStruct((B, S * N), f32),
                  jax.ShapeDtypeStruct((B, S * N), f32)],
        scratch_types=scratch,
    )
    def sc_body(lengthF, kF, qF, dqF, pF, dpF, mF,
                w1s, be1s, we2s, be2s, wn1s, bn1s, wn2s, bn2s,
                hqF, hpF,
                Lb, Kb, qb, dqb, pb, dpb, mb, cst, crt, nbt,
                w1b, be1b, we2b, be2b, wn1b, bn1b, wn2b, bn2b, oq, op_):
        i32 = jnp.int32
        wid = jax.lax.axis_index("s") * 2 + jax.lax.axis_index("c")
        lanei = jax.lax.iota(i32, _LANES)
        idxc = [jnp.full((_LANES,), l, i32) for l in range(_LANES)]
        perms = [lanei ^ kx for kx in (1, 2, 4, 8)]
        mask0 = lanei == 0

        pltpu.sync_copy(w1s, w1b)
        pltpu.sync_copy(be1s, be1b)
        pltpu.sync_copy(we2s, we2b)
        pltpu.sync_copy(be2s, be2b)
        pltpu.sync_copy(wn1s, wn1b)
        pltpu.sync_copy(bn1s, bn1b)
        pltpu.sync_copy(wn2s, wn2b)
        pltpu.sync_copy(bn2s, bn2b)
        _round_region(w1b, 32)
        _round_region(we2b, 128)
        _round_region(wn1b, 140)
        _round_region(wn2b, 4)

        def wrow(buf, base):
            return (buf[pl.ds(base, 16)], buf[pl.ds(base + 16, 16)])
        w1v = [[wrow(w1b, br * 256 + r * 32) for r in range(8)]
               for br in range(2)]
        be1v = [wrow(be1b, br * 32) for br in range(2)]
        bn1v = [wrow(bn1b, br * 32) for br in range(2)]
        be2v = [wrow(be2b, br * 32) for br in range(2)]
        wn2v = [wrow(wn2b, br * 32) for br in range(2)]
        bn2v = [bn2b[pl.ds(br * 16, 16)] for br in range(2)]

        for tloc in range(BPW):
            b = wid * BPW + tloc
            pltpu.sync_copy(lengthF.at[b], Lb)
            pltpu.sync_copy(kF.at[b], Kb)
            pltpu.sync_copy(qF.at[b], qb)
            pltpu.sync_copy(dqF.at[b], dqb)
            pltpu.sync_copy(pF.at[b], pb)
            pltpu.sync_copy(dpF.at[b], dpb)
            pltpu.sync_copy(mF.at[b], mb)
            _round_region(Lb, N * N // 16)
            _round_region(Kb, N * N // 16)
            _round_region(qb, S * N // 16)
            _round_region(dqb, S * N // 16)
            _round_region(pb, S * N // 16)
            _round_region(dpb, S * N // 16)
            _round_region(mb, N // 16)

            def tab_body(j, carry):
                jc = (j // 16) * 16
                lanev = jnp.broadcast_to((j - jc).astype(i32), (_LANES,))
                ms = _gat(mb[pl.ds(jc, 16)], lanev)
                feats = []
                for buf in (qb, dqb, pb, dpb):
                    feats.append([_gat(buf[pl.ds(s * N + jc, 16)], lanev)
                                  for s in range(S)])
                for sb in range(NSB):
                    br, s = sb // 2, sb % 2
                    f1 = feats[2 * br][s]
                    f2 = feats[2 * br + 1][s]
                    for half in range(2):
                        o = sb * (N * 32) + j * 32 + half * 16
                        cs = (f1 * w1v[br][0][half] + f2 * w1v[br][1][half]
                              + ms * w1v[br][2][half])
                        cr = (f1 * w1v[br][3][half] + f2 * w1v[br][4][half]
                              + ms * w1v[br][5][half] + be1v[br][half])
                        nb = (f1 * _w3(wn1b, br, 0, half)
                              + f2 * _w3(wn1b, br, 1, half)
                              + ms * _w3(wn1b, br, 2, half) + bn1v[br][half])
                        cst[pl.ds(o, 16)] = cs
                        crt[pl.ds(o, 16)] = cr
                        nbt[pl.ds(o, 16)] = nb
                return carry
            jax.lax.fori_loop(0, N, tab_body, 0)

            wl = [[w1v[br][6][h] for h in range(2)] for br in range(2)]
            wk = [[w1v[br][7][h] for h in range(2)] for br in range(2)]

            def i_body(i, oacc):
                oacc = list(oacc)
                base = [[crt[pl.ds(sb * (N * 32) + i * 32 + half * 16, 16)]
                         for half in range(2)] for sb in range(NSB)]
                z = jnp.zeros((_LANES,), f32)
                acc0 = [z, z, z, z]
                acc1 = [z, z, z, z]

                def jc_body(jc, accs):
                    a0, a1 = list(accs[0]), list(accs[1])
                    jb = jc * 16
                    lv = Lb[pl.ds(i * N + jb, 16)]
                    kv = Kb[pl.ds(i * N + jb, 16)]
                    for lane in range(16):
                        lbv = _gat(lv, idxc[lane])
                        kbv = _gat(kv, idxc[lane])
                        jo = (jb + lane) * 32
                        for sb in range(NSB):
                            br = sb // 2
                            co = sb * (N * 32) + jo
                            t0 = (cst[pl.ds(co, 16)] + base[sb][0]
                                  + lbv * wl[br][0] + kbv * wk[br][0])
                            t1 = (cst[pl.ds(co + 16, 16)] + base[sb][1]
                                  + lbv * wl[br][1] + kbv * wk[br][1])
                            r0, r1 = _rnd2(jnp.maximum(t0, 0.),
                                           jnp.maximum(t1, 0.))
                            a0[sb] = a0[sb] + r0
                            a1[sb] = a1[sb] + r1
                    return (tuple(a0), tuple(a1))

                acc = jax.lax.fori_loop(0, N // 16, jc_body,
                                        (tuple(acc0), tuple(acc1)))

                for sb in range(NSB):
                    br, s = sb // 2, sb % 2
                    ag0 = be2v[br][0] * float(N)
                    ag1 = be2v[br][1] * float(N)
                    for h in range(32):
                        ahs = _gat(acc[h // 16][sb], idxc[h % 16])
                        wo = br * 1024 + h * 32
                        ag0 = ag0 + ahs * we2b[pl.ds(wo, 16)]
                        ag1 = ag1 + ahs * we2b[pl.ds(wo + 16, 16)]
                    ag0, ag1 = _rnd2(ag0, ag1)
                    no = sb * (N * 32) + i * 32
                    g0 = nbt[pl.ds(no, 16)]
                    g1 = nbt[pl.ds(no + 16, 16)]
                    for kk in range(32):
                        aks = _gat(ag0 if kk < 16 else ag1, idxc[kk % 16])
                        wo = br * 1120 + (3 + kk) * 32
                        g0 = g0 + aks * wn1b[pl.ds(wo, 16)]
                        g1 = g1 + aks * wn1b[pl.ds(wo + 16, 16)]
                    g0, g1 = _rnd2(jnp.maximum(g0, 0.), jnp.maximum(g1, 0.))
                    u = g0 * wn2v[br][0] + g1 * wn2v[br][1]
                    for pv in perms:
                        u = u + _gat(u, pv)
                    val = u + bn2v[br]
                    lmask = lanei == jnp.broadcast_to(i % 16, (_LANES,))
                    oacc[sb] = jnp.where(lmask, val, oacc[sb])
                    obuf = oq if br == 0 else op_
                    obuf[pl.ds(s * N + i - (i % 16), 16)] = oacc[sb]
                return tuple(oacc)
            z16 = jnp.zeros((_LANES,), f32)
            jax.lax.fori_loop(0, N, i_body, (z16, z16, z16, z16))

            pltpu.sync_copy(oq, hqF.at[b])
            pltpu.sync_copy(op_, hpF.at[b])

    return sc_body(*args)


def _w3(wn1b, br, row, half):
    return wn1b[pl.ds(br * 1120 + row * 32 + half * 16, 16)]


def kernel(q, p, dq, dp, m, t, dt, length, k,
           Wqe1, bqe1, Wqe2, bqe2, Wqn1, bqn1, Wqn2, bqn2,
           Wpe1, bpe1, Wpe2, bpe2, Wpn1, bpn1, Wpn2, bpn2):
    B, N, S = q.shape
    f32 = jnp.float32
    lengthF = length.reshape(B, N * N)
    kF = k.reshape(B, N * N)
    qF = jnp.swapaxes(q, 1, 2).reshape(B, S * N)
    dqF = jnp.swapaxes(dq, 1, 2).reshape(B, S * N)
    pF = jnp.swapaxes(p, 1, 2).reshape(B, S * N)
    dpF = jnp.swapaxes(dp, 1, 2).reshape(B, S * N)
    mF = m[..., 0]
    w1s = jnp.stack([Wqe1, Wpe1]).reshape(-1)
    be1s = jnp.stack([bqe1, bpe1]).reshape(-1)
    we2s = jnp.stack([Wqe2, Wpe2]).reshape(-1)
    be2s = jnp.stack([bqe2, bpe2]).reshape(-1)
    wn1s = jnp.stack([Wqn1, Wpn1]).reshape(-1)
    bn1s = jnp.stack([bqn1, bpn1]).reshape(-1)
    wn2s = jnp.stack([Wqn2[:, 0], Wpn2[:, 0]]).reshape(-1)
    bn2s = jnp.stack([jnp.broadcast_to(bqn2, (16,)),
                      jnp.broadcast_to(bpn2, (16,))]).reshape(-1)
    hqF, hpF = _sc_call(B, N, S, f32,
                        (lengthF, kF, qF, dqF, pF, dpF, mF,
                         w1s, be1s, we2s, be2s, wn1s, bn1s, wn2s, bn2s))
    hq = jnp.swapaxes(hqF.reshape(B, S, N), 1, 2)
    hp = jnp.swapaxes(hpF.reshape(B, S, N), 1, 2)
    return hq, hp


# speedup vs baseline: 1.2223x; 1.2223x over previous
import jax
import jax.numpy as jnp
from jax import lax
from jax.experimental import pallas as pl
from jax.experimental.pallas import tpu as pltpu


def _dot(a, b, ca, cb):
    return lax.dot_general(a, b, ((( ca,), (cb,)), ((), ())),
                           precision=lax.Precision.HIGHEST,
                           preferred_element_type=jnp.float32)


def _bf(x):
    return x.astype(jnp.bfloat16).astype(jnp.float32)


def _branch(LT, KT, xT, We1T, be1, We2T, be2, Wn1T, bn1, Wn2T, bn2, n):
    xTb = _bf(xT)
    cs2 = _dot(xTb, _bf(We1T[:, 0:3]), 0, 1)
    baseT = _dot(_bf(We1T[:, 3:6]), xTb, 1, 0) + be1
    wlp = jnp.broadcast_to(_bf(We1T[:, 6:7]), baseT.shape)
    wkp = jnp.broadcast_to(_bf(We1T[:, 7:8]), baseT.shape)
    a3 = (LT[:, None, :] * wlp[None] + KT[:, None, :] * wkp[None]
          + cs2[:, :, None] + baseT[None])
    agg0T = _bf(jnp.maximum(a3, 0.0)).sum(axis=0)
    aggT = _dot(_bf(We2T), agg0T, 1, 0) + n * be2
    gT = jnp.maximum(_dot(_bf(Wn1T[:, 0:3]), xTb, 1, 0)
                     + _dot(_bf(Wn1T[:, 3:]), _bf(aggT), 1, 0) + bn1, 0.0)
    return _dot(_bf(Wn2T), _bf(gT), 1, 0) + bn2


def _body(LT_ref, KT_ref, qT_ref, dqT_ref, pT_ref, dpT_ref, mT_ref,
          We1Tq_ref, be1q_ref, We2Tq_ref, be2q_ref,
          Wn1Tq_ref, bn1q_ref, Wn2Tq_ref, bn2q_ref,
          We1Tp_ref, be1p_ref, We2Tp_ref, be2p_ref,
          Wn1Tp_ref, bn1p_ref, Wn2Tp_ref, bn2p_ref,
          hqT_ref, hpT_ref):
    LT = _bf(LT_ref[0])
    KT = _bf(KT_ref[0])
    mrow = mT_ref[0]
    n = LT.shape[0]
    s_count = qT_ref.shape[1]
    for s in range(s_count):
        xTq = jnp.concatenate([qT_ref[0, s:s + 1, :], dqT_ref[0, s:s + 1, :],
                               mrow], axis=0)
        xTp = jnp.concatenate([pT_ref[0, s:s + 1, :], dpT_ref[0, s:s + 1, :],
                               mrow], axis=0)
        outq = _branch(LT, KT, xTq, We1Tq_ref[...], be1q_ref[...],
                       We2Tq_ref[...], be2q_ref[...], Wn1Tq_ref[...],
                       bn1q_ref[...], Wn2Tq_ref[...], bn2q_ref[...], n)
        outp = _branch(LT, KT, xTp, We1Tp_ref[...], be1p_ref[...],
                       We2Tp_ref[...], be2p_ref[...], Wn1Tp_ref[...],
                       bn1p_ref[...], Wn2Tp_ref[...], bn2p_ref[...], n)
        hqT_ref[0, s, :] = outq[0]
        hpT_ref[0, s, :] = outp[0]


def _kernel_tc(q, p, dq, dp, m, t, dt, length, k,
               Wqe1, bqe1, Wqe2, bqe2, Wqn1, bqn1, Wqn2, bqn2,
               Wpe1, bpe1, Wpe2, bpe2, Wpn1, bpn1, Wpn2, bpn2):
    B, N, S = q.shape
    H = Wqe1.shape[1]
    f32 = jnp.float32

    LT = jnp.swapaxes(length, 1, 2)
    KT = jnp.swapaxes(k, 1, 2)
    qT = jnp.swapaxes(q, 1, 2)
    dqT = jnp.swapaxes(dq, 1, 2)
    pT = jnp.swapaxes(p, 1, 2)
    dpT = jnp.swapaxes(dp, 1, 2)
    mT = jnp.swapaxes(m, 1, 2)

    wargs = (Wqe1.T, bqe1[:, None], Wqe2.T, bqe2[:, None],
             Wqn1.T, bqn1[:, None], Wqn2.T, bqn2[:, None],
             Wpe1.T, bpe1[:, None], Wpe2.T, bpe2[:, None],
             Wpn1.T, bpn1[:, None], Wpn2.T, bpn2[:, None])

    def bspec(shape3):
        return pl.BlockSpec(shape3, lambda b: (b, 0, 0))

    def wspec(arr):
        sh = arr.shape
        return pl.BlockSpec(sh, lambda b: tuple(0 for _ in sh))

    grid_spec = pl.GridSpec(
        grid=(B,),
        in_specs=[bspec((1, N, N)), bspec((1, N, N)),
                  bspec((1, S, N)), bspec((1, S, N)),
                  bspec((1, S, N)), bspec((1, S, N)),
                  bspec((1, 1, N))] + [wspec(w) for w in wargs],
        out_specs=[bspec((1, S, N)), bspec((1, S, N))],
    )

    hqT, hpT = pl.pallas_call(
        _body,
        grid_spec=grid_spec,
        out_shape=[jax.ShapeDtypeStruct((B, S, N), f32),
                   jax.ShapeDtypeStruct((B, S, N), f32)],
        compiler_params=pltpu.CompilerParams(
            dimension_semantics=("arbitrary",)),
    )(LT, KT, qT, dqT, pT, dpT, mT, *wargs)

    return jnp.swapaxes(hqT, 1, 2), jnp.swapaxes(hpT, 1, 2)


import functools
from jax.experimental.pallas import tpu_sc as plsc

_LANES = 16


def _rnd1(x):
    u = jax.lax.bitcast_convert_type(x, jnp.int32)
    r = (u + 32768) & jnp.int32(-65536)
    return jax.lax.bitcast_convert_type(r, jnp.float32)


def _rnd2(a, b):
    return _rnd1(a), _rnd1(b)


def _gat(vec, idxv):
    return vec.at[idxv].get(mode="promise_in_bounds")


def _round_region(ref, nchunks):
    def body(c, carry):
        o = c * 32
        a = ref[pl.ds(o, 16)]
        b = ref[pl.ds(o + 16, 16)]
        ar, br = _rnd2(a, b)
        ref[pl.ds(o, 16)] = ar
        ref[pl.ds(o + 16, 16)] = br
        return carry
    jax.lax.fori_loop(0, nchunks // 2, body, 0)


def _sc_call(B, N, S, f32, args):
    NSB = 4
    mesh = plsc.VectorSubcoreMesh(core_axis_name="c", subcore_axis_name="s")
    NW = 32
    BPW = B // NW

    scratch = [
        pltpu.VMEM((N * N,), f32),
        pltpu.VMEM((N * N,), f32),
        pltpu.VMEM((S * N,), f32),
        pltpu.VMEM((S * N,), f32),
        pltpu.VMEM((S * N,), f32),
        pltpu.VMEM((S * N,), f32),
        pltpu.VMEM((N,), f32),
        pltpu.VMEM((NSB * N * 32,), f32),
        pltpu.VMEM((NSB * N * 32,), f32),
        pltpu.VMEM((NSB * N * 32,), f32),
        pltpu.VMEM((512,), f32),
        pltpu.VMEM((64,), f32),
        pltpu.VMEM((2048,), f32),
        pltpu.VMEM((64,), f32),
        pltpu.VMEM((2240,), f32),
        pltpu.VMEM((64,), f32),
        pltpu.VMEM((64,), f32),
        pltpu.VMEM((32,), f32),
        pltpu.VMEM((S * N,), f32),
        pltpu.VMEM((S * N,), f32),
    ]

    @functools.partial(
        pl.kernel, mesh=mesh,
        out_type=[jax.ShapeDtypeStruct((B, S * N), f32),
                  jax.ShapeDtypeStruct((B, S * N), f32)],
        scratch_types=scratch,
    )
    def sc_body(lengthF, kF, qF, dqF, pF, dpF, mF,
                w1s, be1s, we2s, be2s, wn1s, bn1s, wn2s, bn2s,
                hqF, hpF,
                Lb, Kb, qb, dqb, pb, dpb, mb, cst, crt, nbt,
                w1b, be1b, we2b, be2b, wn1b, bn1b, wn2b, bn2b, oq, op_):
        i32 = jnp.int32
        wid = jax.lax.axis_index("s") * 2 + jax.lax.axis_index("c")
        lanei = jax.lax.iota(i32, _LANES)
        idxc = [jnp.full((_LANES,), l, i32) for l in range(_LANES)]
        perms = [lanei ^ kx for kx in (1, 2, 4, 8)]
        mask0 = lanei == 0

        pltpu.sync_copy(w1s, w1b)
        pltpu.sync_copy(be1s, be1b)
        pltpu.sync_copy(we2s, we2b)
        pltpu.sync_copy(be2s, be2b)
        pltpu.sync_copy(wn1s, wn1b)
        pltpu.sync_copy(bn1s, bn1b)
        pltpu.sync_copy(wn2s, wn2b)
        pltpu.sync_copy(bn2s, bn2b)
        _round_region(w1b, 32)
        _round_region(we2b, 128)
        _round_region(wn1b, 140)
        _round_region(wn2b, 4)

        def wrow(buf, base):
            return (buf[pl.ds(base, 16)], buf[pl.ds(base + 16, 16)])
        w1v = [[wrow(w1b, br * 256 + r * 32) for r in range(8)]
               for br in range(2)]
        be1v = [wrow(be1b, br * 32) for br in range(2)]
        bn1v = [wrow(bn1b, br * 32) for br in range(2)]
        be2v = [wrow(be2b, br * 32) for br in range(2)]
        wn2v = [wrow(wn2b, br * 32) for br in range(2)]
        bn2v = [bn2b[pl.ds(br * 16, 16)] for br in range(2)]

        for tloc in range(BPW):
            b = wid * BPW + tloc
            pltpu.sync_copy(lengthF.at[b], Lb)
            pltpu.sync_copy(kF.at[b], Kb)
            pltpu.sync_copy(qF.at[b], qb)
            pltpu.sync_copy(dqF.at[b], dqb)
            pltpu.sync_copy(pF.at[b], pb)
            pltpu.sync_copy(dpF.at[b], dpb)
            pltpu.sync_copy(mF.at[b], mb)
            _round_region(Lb, N * N // 16)
            _round_region(Kb, N * N // 16)
            _round_region(qb, S * N // 16)
            _round_region(dqb, S * N // 16)
            _round_region(pb, S * N // 16)
            _round_region(dpb, S * N // 16)
            _round_region(mb, N // 16)

            def tab_body(j, carry):
                jc = (j // 16) * 16
                lanev = jnp.broadcast_to((j - jc).astype(i32), (_LANES,))
                ms = _gat(mb[pl.ds(jc, 16)], lanev)
                feats = []
                for buf in (qb, dqb, pb, dpb):
                    feats.append([_gat(buf[pl.ds(s * N + jc, 16)], lanev)
                                  for s in range(S)])
                for sb in range(NSB):
                    br, s = sb // 2, sb % 2
                    f1 = feats[2 * br][s]
                    f2 = feats[2 * br + 1][s]
                    for half in range(2):
                        o = sb * (N * 32) + j * 32 + half * 16
                        cs = (f1 * w1v[br][0][half] + f2 * w1v[br][1][half]
                              + ms * w1v[br][2][half])
                        cr = (f1 * w1v[br][3][half] + f2 * w1v[br][4][half]
                              + ms * w1v[br][5][half] + be1v[br][half])
                        nb = (f1 * _w3(wn1b, br, 0, half)
                              + f2 * _w3(wn1b, br, 1, half)
                              + ms * _w3(wn1b, br, 2, half) + bn1v[br][half])
                        cst[pl.ds(o, 16)] = cs
                        crt[pl.ds(o, 16)] = cr
                        nbt[pl.ds(o, 16)] = nb
                return carry
            jax.lax.fori_loop(0, N, tab_body, 0)

            wl = [[w1v[br][6][h] for h in range(2)] for br in range(2)]
            wk = [[w1v[br][7][h] for h in range(2)] for br in range(2)]

            def i_body(i, oacc):
                oacc = list(oacc)
                base = [[crt[pl.ds(sb * (N * 32) + i * 32 + half * 16, 16)]
                         for half in range(2)] for sb in range(NSB)]
                z = jnp.zeros((_LANES,), f32)
                acc0 = [z, z, z, z]
                acc1 = [z, z, z, z]

                def jc_body(jc, accs):
                    a0, a1 = list(accs[0]), list(accs[1])
                    jb = jc * 16
                    lv = Lb[pl.ds(i * N + jb, 16)]
                    kv = Kb[pl.ds(i * N + jb, 16)]
                    for lane in range(16):
                        lbv = _gat(lv, idxc[lane])
                        kbv = _gat(kv, idxc[lane])
                        jo = (jb + lane) * 32
                        for sb in range(NSB):
                            br = sb // 2
                            co = sb * (N * 32) + jo
                            t0 = (cst[pl.ds(co, 16)] + base[sb][0]
                                  + lbv * wl[br][0] + kbv * wk[br][0])
                            t1 = (cst[pl.ds(co + 16, 16)] + base[sb][1]
                                  + lbv * wl[br][1] + kbv * wk[br][1])
                            r0, r1 = _rnd2(jnp.maximum(t0, 0.),
                                           jnp.maximum(t1, 0.))
                            a0[sb] = a0[sb] + r0
                            a1[sb] = a1[sb] + r1
                    return (tuple(a0), tuple(a1))

                acc = jax.lax.fori_loop(0, N // 16, jc_body,
                                        (tuple(acc0), tuple(acc1)))

                for sb in range(NSB):
                    br, s = sb // 2, sb % 2
                    ag0 = be2v[br][0] * float(N)
                    ag1 = be2v[br][1] * float(N)
                    for h in range(32):
                        ahs = _gat(acc[h // 16][sb], idxc[h % 16])
                        wo = br * 1024 + h * 32
                        ag0 = ag0 + ahs * we2b[pl.ds(wo, 16)]
                        ag1 = ag1 + ahs * we2b[pl.ds(wo + 16, 16)]
                    ag0, ag1 = _rnd2(ag0, ag1)
                    no = sb * (N * 32) + i * 32
                    g0 = nbt[pl.ds(no, 16)]
                    g1 = nbt[pl.ds(no + 16, 16)]
                    for kk in range(32):
                        aks = _gat(ag0 if kk < 16 else ag1, idxc[kk % 16])
                        wo = br * 1120 + (3 + kk) * 32
                        g0 = g0 + aks * wn1b[pl.ds(wo, 16)]
                        g1 = g1 + aks * wn1b[pl.ds(wo + 16, 16)]
                    g0, g1 = _rnd2(jnp.maximum(g0, 0.), jnp.maximum(g1, 0.))
                    u = g0 * wn2v[br][0] + g1 * wn2v[br][1]
                    for pv in perms:
                        u = u + _gat(u, pv)
                    val = u + bn2v[br]
                    lmask = lanei == jnp.broadcast_to(i % 16, (_LANES,))
                    oacc[sb] = jnp.where(lmask, val, oacc[sb])
                    obuf = oq if br == 0 else op_
                    obuf[pl.ds(s * N + i - (i % 16), 16)] = oacc[sb]
                return tuple(oacc)
            z16 = jnp.zeros((_LANES,), f32)
            jax.lax.fori_loop(0, N, i_body, (z16, z16, z16, z16))

            pltpu.sync_copy(oq, hqF.at[b])
            pltpu.sync_copy(op_, hpF.at[b])

    return sc_body(*args)


def _w3(wn1b, br, row, half):
    return wn1b[pl.ds(br * 1120 + row * 32 + half * 16, 16)]


def kernel(q, p, dq, dp, m, t, dt, length, k,
           Wqe1, bqe1, Wqe2, bqe2, Wqn1, bqn1, Wqn2, bqn2,
           Wpe1, bpe1, Wpe2, bpe2, Wpn1, bpn1, Wpn2, bpn2):
    B, N, S = q.shape
    f32 = jnp.float32
    lengthF = length.reshape(B, N * N)
    kF = k.reshape(B, N * N)
    qF = jnp.swapaxes(q, 1, 2).reshape(B, S * N)
    dqF = jnp.swapaxes(dq, 1, 2).reshape(B, S * N)
    pF = jnp.swapaxes(p, 1, 2).reshape(B, S * N)
    dpF = jnp.swapaxes(dp, 1, 2).reshape(B, S * N)
    mF = m[..., 0]
    w1s = jnp.stack([Wqe1, Wpe1]).reshape(-1)
    be1s = jnp.stack([bqe1, bpe1]).reshape(-1)
    we2s = jnp.stack([Wqe2, Wpe2]).reshape(-1)
    be2s = jnp.stack([bqe2, bpe2]).reshape(-1)
    wn1s = jnp.stack([Wqn1, Wpn1]).reshape(-1)
    bn1s = jnp.stack([bqn1, bpn1]).reshape(-1)
    wn2s = jnp.stack([Wqn2[:, 0], Wpn2[:, 0]]).reshape(-1)
    bn2s = jnp.stack([jnp.broadcast_to(bqn2, (16,)),
                      jnp.broadcast_to(bpn2, (16,))]).reshape(-1)
    hqF, hpF = _sc_call(B, N, S, f32,
                        (lengthF, kF, qF, dqF, pF, dpF, mF,
                         w1s, be1s, we2s, be2s, wn1s, bn1s, wn2s, bn2s))
    hq = jnp.swapaxes(hqF.reshape(B, S, N), 1, 2)
    hp = jnp.swapaxes(hpF.reshape(B, S, N), 1, 2)
    return hq, hp
